# baseline (speedup 1.0000x reference)
import functools

import jax
import jax.numpy as jnp
from jax import lax
from jax.experimental import pallas as pl
from jax.experimental.pallas import tpu as pltpu
from jax.experimental.pallas import tpu_sc as plsc

_LANES = 16
_IDX_CHUNK = 128


def _worker_id():
    return lax.axis_index("s") * 2 + lax.axis_index("c")


def _take16(vec, idx):
    return lax.gather(
        vec, idx[:, None],
        dimension_numbers=lax.GatherDimensionNumbers(
            offset_dims=(), collapsed_slice_dims=(0,), start_index_map=(0,)),
        slice_sizes=(1,),
        mode=lax.GatherScatterMode.PROMISE_IN_BOUNDS)


def _group_starts(list_len):
    starts = list(range(0, list_len - _LANES + 1, _LANES))
    if starts[-1] + _LANES < list_len:
        starts.append(list_len - _LANES)
    return starts


@functools.lru_cache(maxsize=None)
def _make_format_kernel(batch, list_len, vocab, dim):
    info = plsc.get_sparse_core_info()
    nw = info.num_cores * info.num_subcores
    users_per_w = batch // nw
    n_tile_cols = vocab // _IDX_CHUNK
    vocab_tail = vocab - n_tile_cols * _IDX_CHUNK
    full_cols = n_tile_cols
    cb_w = _IDX_CHUNK * 2
    n_batch = full_cols // 2
    units = (n_batch // nw) & ~1
    rem_batches = n_batch - units * nw
    mesh = plsc.VectorSubcoreMesh(core_axis_name="c", subcore_axis_name="s")

    @functools.partial(
        pl.kernel,
        out_type=(
            jax.ShapeDtypeStruct((vocab * dim,), jnp.float32),
            jax.ShapeDtypeStruct((batch * dim,), jnp.float32),
            jax.ShapeDtypeStruct((batch * list_len,), jnp.int32),
        ),
        mesh=mesh,
        compiler_params=pltpu.CompilerParams(needs_layout_passes=False),
        scratch_types=[
            pltpu.VMEM((dim, cb_w), jnp.float32),
            pltpu.VMEM((dim, cb_w), jnp.float32),
            pltpu.VMEM((cb_w * dim,), jnp.float32),
            pltpu.VMEM((cb_w * dim,), jnp.float32),
            pltpu.VMEM((dim, _IDX_CHUNK), jnp.float32),
            pltpu.VMEM((dim, _IDX_CHUNK), jnp.float32),
            pltpu.VMEM((dim, vocab_tail), jnp.float32) if vocab_tail
            else pltpu.VMEM((dim, _IDX_CHUNK), jnp.float32),
            pltpu.VMEM((max(vocab_tail, 1) * dim,), jnp.float32),
            pltpu.VMEM((users_per_w * dim,), jnp.float32),
            pltpu.VMEM((list_len, _IDX_CHUNK), jnp.int32),
            pltpu.VMEM((users_per_w * list_len,), jnp.int32),
            pltpu.VMEM((users_per_w,), jnp.int32),
            pltpu.SemaphoreType.DMA,
            pltpu.SemaphoreType.DMA,
            pltpu.SemaphoreType.DMA,
            pltpu.SemaphoreType.DMA,
        ],
    )
    def ka(itab_t, ids_t, uid_hbm, utab_t,
           it_flat, uemb_flat, ids_flat,
           vb0, vb1, tb0, tb1, ub0, ub1, pb, ptb, ueb, ib, iob, uvm,
           si0, si1, so0, so1):
        vbs = (vb0, vb1)
        tbs = (tb0, tb1)
        ubs = (ub0, ub1)
        sis = (si0, si1)
        sos = (so0, so1)
        wid = _worker_id()
        rbase = wid * units * cb_w

        def in_desc(k, b):
            return pltpu.make_async_copy(
                itab_t.at[:, pl.ds(rbase + k * cb_w, cb_w)],
                vbs[b], sis[b])

        def out_desc(k, b):
            return pltpu.make_async_copy(
                tbs[b],
                it_flat.at[pl.ds((rbase + k * cb_w) * dim, cb_w * dim)],
                sos[b])

        iota = lax.iota(jnp.int32, _LANES)
        perms = [((iota + s) & (_LANES - 1)) for s in range(_LANES)]

        def transpose_block(src, dst, n_cols):
            def cols16(i, carry):
                colidx = iota + i * _LANES
                base = colidx * dim
                for d0 in range(0, dim, _LANES):
                    for s in range(_LANES):
                        rowidx = perms[s] + d0
                        v = plsc.load_gather(src, [rowidx, colidx])
                        plsc.store_scatter(dst, [base + rowidx], v)
                return carry
            lax.fori_loop(0, n_cols // _LANES, cols16, 0)

        in_desc(0, 0).start()

        def col_body(k2, carry):
            for b in range(2):
                k = k2 * 2 + b
                in_desc(k, b).wait()

                @pl.when(k + 1 < units)
                def _pref():
                    in_desc(k + 1, 1 - b).start()

                @pl.when(k >= 2)
                def _drain():
                    out_desc(k - 2, b).wait()

                transpose_block(vbs[b], tbs[b], cb_w)
                out_desc(k, b).start()
            return carry

        lax.fori_loop(0, units // 2, col_body, 0)
        out_desc(units - 2, 0).wait()
        out_desc(units - 1, 1).wait()

        @pl.when(wid == nw - 1)
        def _tail():
            r0 = nw * units * cb_w
            for j in range(rem_batches):
                pltpu.async_copy(
                    itab_t.at[:, pl.ds(r0 + j * cb_w, cb_w)], vb0, si0).wait()
                transpose_block(vb0, tb0, cb_w)
                pltpu.async_copy(
                    tb0, it_flat.at[pl.ds((r0 + j * cb_w) * dim, cb_w * dim)],
                    so0).wait()
            if vocab_tail:
                pltpu.async_copy(
                    itab_t.at[:, pl.ds(n_tile_cols * _IDX_CHUNK, vocab_tail)],
                    pb, si0).wait()
                transpose_block(pb, ptb, vocab_tail)
                pltpu.async_copy(
                    ptb,
                    it_flat.at[pl.ds(n_tile_cols * _IDX_CHUNK * dim,
                                     vocab_tail * dim)],
                    so0).wait()

        ubase = wid * users_per_w
        pltpu.sync_copy(uid_hbm.at[pl.ds(ubase, users_per_w)], uvm)

        def ufetch_desc(uid_scalar, b):
            col = pl.multiple_of((uid_scalar // _IDX_CHUNK) * _IDX_CHUNK,
                                 _IDX_CHUNK)
            return pltpu.make_async_copy(
                utab_t.at[:, pl.ds(col, _IDX_CHUNK)], ubs[b], sis[b])

        def user_body(u2, carry):
            uvec = uvm[pl.ds(u2 * _LANES, _LANES)]
            uids = [uvec[j] for j in range(_LANES)]
            ufetch_desc(uids[0], 0).start()
            for j in range(_LANES):
                b = j % 2
                ufetch_desc(uids[j], b).wait()
                if j + 1 < _LANES:
                    ufetch_desc(uids[j + 1], 1 - b).start()
                lane = uids[j] % _IDX_CHUNK
                u = u2 * _LANES + j
                for d0 in range(0, dim, _LANES):
                    ridx = lax.iota(jnp.int32, _LANES) + d0
                    cidx = jnp.full((_LANES,), lane, jnp.int32)
                    v = plsc.load_gather(ubs[b], [ridx, cidx])
                    ueb[pl.ds(u * dim + d0, _LANES)] = v
            return carry

        lax.fori_loop(0, users_per_w // _LANES, user_body, 0)
        pltpu.sync_copy(ueb, uemb_flat.at[pl.ds(ubase * dim,
                                                users_per_w * dim)])

        pltpu.sync_copy(ids_t.at[:, pl.ds(ubase, users_per_w)], ib)

        def ids_body(u, carry):
            for l0 in _group_starts(list_len):
                ridx = lax.iota(jnp.int32, _LANES) + l0
                cidx = jnp.full((_LANES,), u, jnp.int32)
                v = plsc.load_gather(ib, [ridx, cidx])
                iob[pl.ds(u * list_len + l0, _LANES)] = v
            return carry

        lax.fori_loop(0, users_per_w, ids_body, 0)
        pltpu.sync_copy(
            iob, ids_flat.at[pl.ds(ubase * list_len, users_per_w * list_len)])

    return ka


@functools.lru_cache(maxsize=None)
def _make_score_kernel(batch, list_len, vocab, dim):
    info = plsc.get_sparse_core_info()
    nw = info.num_cores * info.num_subcores
    users_per_w = batch // nw
    sub_sizes = []
    rem = list_len
    while rem > 0:
        s = min(_IDX_CHUNK, rem)
        sub_sizes.append(s)
        rem -= s

    mesh = plsc.VectorSubcoreMesh(core_axis_name="c", subcore_axis_name="s")

    @functools.partial(
        pl.kernel,
        out_type=jax.ShapeDtypeStruct((batch * list_len,), jnp.float32),
        mesh=mesh,
        compiler_params=pltpu.CompilerParams(
            needs_layout_passes=False, use_tc_tiling_on_sc=False),
        scratch_types=[
            pltpu.VMEM((users_per_w, dim), jnp.float32),
            pltpu.VMEM((users_per_w * list_len,), jnp.int32),
            pltpu.VMEM((list_len, dim), jnp.float32),
            pltpu.VMEM((list_len, dim), jnp.float32),
            pltpu.VMEM((users_per_w * list_len,), jnp.float32),
            pltpu.SemaphoreType.DMA,
            pltpu.SemaphoreType.DMA,
            pltpu.SemaphoreType.DMA,
        ],
    )
    def kb(uemb_hbm, iid_hbm, itab_hbm, out_hbm,
           urows, iidx, rows0, rows1, scores, sem0, sem1, semu):
        rows = (rows0, rows1)
        sems = (sem0, sem1)
        wid = _worker_id()
        ubase = wid * users_per_w

        pltpu.sync_copy(
            uemb_hbm.at[pl.ds(ubase, users_per_w), :], urows)
        pltpu.sync_copy(
            iid_hbm.at[pl.ds(ubase * list_len, users_per_w * list_len)], iidx)

        def gather_descs(u, b):
            off = pl.multiple_of(u * list_len, 8)
            descs = []
            pos = 0
            for s in sub_sizes:
                descs.append(pltpu.make_async_copy(
                    itab_hbm.at[iidx.at[pl.ds(off + pos, s)]],
                    rows[b].at[pl.ds(pos, s)],
                    sems[b]))
                pos += s
            return descs

        def compute(u, rowsb):
            iota = lax.iota(jnp.int32, _LANES)
            u0 = urows[u, pl.ds(0, _LANES)]
            u1 = urows[u, pl.ds(_LANES, _LANES)]
            starts = _group_starts(list_len)
            ridxs = [iota + l0 for l0 in starts]
            accs = [jnp.zeros((_LANES,), jnp.float32) for _ in starts]
            for d in range(dim):
                colidx = (iota + d) & (dim - 1)
                lo = colidx & (_LANES - 1)
                urot = jnp.where(colidx < _LANES,
                                 _take16(u0, lo), _take16(u1, lo))
                for g in range(len(starts)):
                    v = plsc.load_gather(rowsb, [ridxs[g], colidx])
                    accs[g] = accs[g] + v * urot
            for g, l0 in enumerate(starts):
                scores[pl.ds(u * list_len + l0, _LANES)] = accs[g]

        for d in gather_descs(0, 0):
            d.start()

        def body(uu, carry):
            for b in range(2):
                u = uu * 2 + b
                for d in gather_descs(u, b):
                    d.wait()

                @pl.when(u + 1 < users_per_w)
                def _prefetch():
                    for d in gather_descs(u + 1, 1 - b):
                        d.start()

                compute(u, rows[b])
            return carry

        lax.fori_loop(0, users_per_w // 2, body, 0)
        pltpu.sync_copy(
            scores,
            out_hbm.at[pl.ds(ubase * list_len, users_per_w * list_len)])

    return kb


@functools.lru_cache(maxsize=None)
def _make_retile_kernel(batch, list_len):
    info = plsc.get_sparse_core_info()
    nw = info.num_cores * info.num_subcores
    users_per_w = batch // nw
    mesh = plsc.VectorSubcoreMesh(core_axis_name="c", subcore_axis_name="s")

    @functools.partial(
        pl.kernel,
        out_type=jax.ShapeDtypeStruct((list_len, batch), jnp.float32),
        mesh=mesh,
        compiler_params=pltpu.CompilerParams(needs_layout_passes=False),
        scratch_types=[
            pltpu.VMEM((users_per_w * list_len,), jnp.float32),
            pltpu.VMEM((list_len, users_per_w), jnp.float32),
            pltpu.SemaphoreType.DMA,
        ],
    )
    def kc(flat_hbm, out_hbm, sbuf, obuf, sem):
        wid = _worker_id()
        ubase = wid * users_per_w
        pltpu.async_copy(
            flat_hbm.at[pl.ds(ubase * list_len, users_per_w * list_len)],
            sbuf, sem).wait()

        def row(l, carry):
            for u0 in range(0, users_per_w, _LANES):
                idx = (lax.iota(jnp.int32, _LANES) + u0) * list_len + l
                v = plsc.load_gather(sbuf, [idx])
                obuf[l, pl.ds(u0, _LANES)] = v
            return carry

        lax.fori_loop(0, list_len, row, 0)
        pltpu.async_copy(
            obuf, out_hbm.at[:, pl.ds(ubase, users_per_w)], sem).wait()

    return kc


def kernel(user_id, item_ids, user_table, item_table):
    batch, list_len = item_ids.shape
    vocab, dim = item_table.shape
    ka = _make_format_kernel(batch, list_len, vocab, dim)
    it_flat, uemb_flat, ids_flat = ka(
        item_table.T, item_ids.T, user_id, user_table.T)
    kb = _make_score_kernel(batch, list_len, vocab, dim)
    scores_flat = kb(uemb_flat.reshape(batch, dim), ids_flat,
                     it_flat.reshape(vocab, dim))
    kc = _make_retile_kernel(batch, list_len)
    return kc(scores_flat).T


# speedup vs baseline: 1.1475x; 1.1475x over previous
import functools

import jax
import jax.numpy as jnp
from jax import lax
from jax.experimental import pallas as pl
from jax.experimental.pallas import tpu as pltpu
from jax.experimental.pallas import tpu_sc as plsc

_LANES = 16
_IDX_CHUNK = 128


def _worker_id():
    return lax.axis_index("s") * 2 + lax.axis_index("c")


def _take16(vec, idx):
    return lax.gather(
        vec, idx[:, None],
        dimension_numbers=lax.GatherDimensionNumbers(
            offset_dims=(), collapsed_slice_dims=(0,), start_index_map=(0,)),
        slice_sizes=(1,),
        mode=lax.GatherScatterMode.PROMISE_IN_BOUNDS)


def _group_starts(list_len):
    starts = list(range(0, list_len - _LANES + 1, _LANES))
    if starts[-1] + _LANES < list_len:
        starts.append(list_len - _LANES)
    return starts


@functools.lru_cache(maxsize=None)
def _make_format_kernel(batch, list_len, vocab, dim):
    info = plsc.get_sparse_core_info()
    nw = info.num_cores * info.num_subcores
    users_per_w = batch // nw
    n_tile_cols = vocab // _IDX_CHUNK
    vocab_tail = vocab - n_tile_cols * _IDX_CHUNK
    full_cols = n_tile_cols
    cb_w = _IDX_CHUNK * 2
    n_batch = full_cols // 2
    units = (n_batch // nw) & ~1
    rem_batches = n_batch - units * nw
    mesh = plsc.VectorSubcoreMesh(core_axis_name="c", subcore_axis_name="s")

    @functools.partial(
        pl.kernel,
        out_type=(
            jax.ShapeDtypeStruct((vocab * dim,), jnp.float32),
            jax.ShapeDtypeStruct((batch * dim,), jnp.float32),
            jax.ShapeDtypeStruct((batch * list_len,), jnp.int32),
        ),
        mesh=mesh,
        compiler_params=pltpu.CompilerParams(needs_layout_passes=False),
        scratch_types=[
            pltpu.VMEM((dim, cb_w), jnp.float32),
            pltpu.VMEM((dim, cb_w), jnp.float32),
            pltpu.VMEM((cb_w * dim,), jnp.float32),
            pltpu.VMEM((cb_w * dim,), jnp.float32),
            pltpu.VMEM((dim, _IDX_CHUNK), jnp.float32),
            pltpu.VMEM((dim, _IDX_CHUNK), jnp.float32),
            pltpu.VMEM((dim, vocab_tail), jnp.float32) if vocab_tail
            else pltpu.VMEM((dim, _IDX_CHUNK), jnp.float32),
            pltpu.VMEM((max(vocab_tail, 1) * dim,), jnp.float32),
            pltpu.VMEM((users_per_w * dim,), jnp.float32),
            pltpu.VMEM((list_len, _IDX_CHUNK), jnp.int32),
            pltpu.VMEM((users_per_w * list_len,), jnp.int32),
            pltpu.VMEM((users_per_w,), jnp.int32),
            pltpu.SemaphoreType.DMA,
            pltpu.SemaphoreType.DMA,
            pltpu.SemaphoreType.DMA,
            pltpu.SemaphoreType.DMA,
        ],
    )
    def ka(itab_t, ids_t, uid_hbm, utab_t,
           it_flat, uemb_flat, ids_flat,
           vb0, vb1, tb0, tb1, ub0, ub1, pb, ptb, ueb, ib, iob, uvm,
           si0, si1, so0, so1):
        vbs = (vb0, vb1)
        tbs = (tb0, tb1)
        ubs = (ub0, ub1)
        sis = (si0, si1)
        sos = (so0, so1)
        wid = _worker_id()
        rbase = wid * units * cb_w

        def in_desc(k, b):
            return pltpu.make_async_copy(
                itab_t.at[:, pl.ds(rbase + k * cb_w, cb_w)],
                vbs[b], sis[b])

        def out_desc(k, b):
            return pltpu.make_async_copy(
                tbs[b],
                it_flat.at[pl.ds((rbase + k * cb_w) * dim, cb_w * dim)],
                sos[b])

        iota = lax.iota(jnp.int32, _LANES)
        perms = [((iota + s) & (_LANES - 1)) for s in range(_LANES)]

        def transpose_block(src, dst, n_cols):
            def cols16(i, carry):
                colidx = iota + i * _LANES
                base = colidx * dim
                pairs = []
                for d0 in range(0, dim, _LANES):
                    for s in range(_LANES):
                        rowidx = perms[s] + d0
                        v = plsc.load_gather(src, [rowidx, colidx])
                        pairs.append((base + rowidx, v))
                for idx, v in pairs:
                    plsc.store_scatter(dst, [idx], v)
                return carry
            lax.fori_loop(0, n_cols // _LANES, cols16, 0)

        in_desc(0, 0).start()

        def col_body(k2, carry):
            for b in range(2):
                k = k2 * 2 + b
                in_desc(k, b).wait()

                @pl.when(k + 1 < units)
                def _pref():
                    in_desc(k + 1, 1 - b).start()

                @pl.when(k >= 2)
                def _drain():
                    out_desc(k - 2, b).wait()

                transpose_block(vbs[b], tbs[b], cb_w)
                out_desc(k, b).start()
            return carry

        lax.fori_loop(0, units // 2, col_body, 0)
        out_desc(units - 2, 0).wait()
        out_desc(units - 1, 1).wait()

        @pl.when(wid == nw - 1)
        def _tail():
            r0 = nw * units * cb_w
            for j in range(rem_batches):
                pltpu.async_copy(
                    itab_t.at[:, pl.ds(r0 + j * cb_w, cb_w)], vb0, si0).wait()
                transpose_block(vb0, tb0, cb_w)
                pltpu.async_copy(
                    tb0, it_flat.at[pl.ds((r0 + j * cb_w) * dim, cb_w * dim)],
                    so0).wait()
            if vocab_tail:
                pltpu.async_copy(
                    itab_t.at[:, pl.ds(n_tile_cols * _IDX_CHUNK, vocab_tail)],
                    pb, si0).wait()
                transpose_block(pb, ptb, vocab_tail)
                pltpu.async_copy(
                    ptb,
                    it_flat.at[pl.ds(n_tile_cols * _IDX_CHUNK * dim,
                                     vocab_tail * dim)],
                    so0).wait()

        ubase = wid * users_per_w
        pltpu.sync_copy(uid_hbm.at[pl.ds(ubase, users_per_w)], uvm)

        def ufetch_desc(uid_scalar, b):
            col = pl.multiple_of((uid_scalar // _IDX_CHUNK) * _IDX_CHUNK,
                                 _IDX_CHUNK)
            return pltpu.make_async_copy(
                utab_t.at[:, pl.ds(col, _IDX_CHUNK)], ubs[b], sis[b])

        def user_body(u2, carry):
            uvec = uvm[pl.ds(u2 * _LANES, _LANES)]
            uids = [uvec[j] for j in range(_LANES)]
            ufetch_desc(uids[0], 0).start()
            for j in range(_LANES):
                b = j % 2
                ufetch_desc(uids[j], b).wait()
                if j + 1 < _LANES:
                    ufetch_desc(uids[j + 1], 1 - b).start()
                lane = uids[j] % _IDX_CHUNK
                u = u2 * _LANES + j
                for d0 in range(0, dim, _LANES):
                    ridx = lax.iota(jnp.int32, _LANES) + d0
                    cidx = jnp.full((_LANES,), lane, jnp.int32)
                    v = plsc.load_gather(ubs[b], [ridx, cidx])
                    ueb[pl.ds(u * dim + d0, _LANES)] = v
            return carry

        lax.fori_loop(0, users_per_w // _LANES, user_body, 0)
        pltpu.sync_copy(ueb, uemb_flat.at[pl.ds(ubase * dim,
                                                users_per_w * dim)])

        pltpu.sync_copy(ids_t.at[:, pl.ds(ubase, users_per_w)], ib)

        def ids_body(u, carry):
            for l0 in _group_starts(list_len):
                ridx = lax.iota(jnp.int32, _LANES) + l0
                cidx = jnp.full((_LANES,), u, jnp.int32)
                v = plsc.load_gather(ib, [ridx, cidx])
                iob[pl.ds(u * list_len + l0, _LANES)] = v
            return carry

        lax.fori_loop(0, users_per_w, ids_body, 0)
        pltpu.sync_copy(
            iob, ids_flat.at[pl.ds(ubase * list_len, users_per_w * list_len)])

    return ka


@functools.lru_cache(maxsize=None)
def _make_score_kernel(batch, list_len, vocab, dim):
    info = plsc.get_sparse_core_info()
    nw = info.num_cores * info.num_subcores
    users_per_w = batch // nw
    sub_sizes = []
    rem = list_len
    while rem > 0:
        s = min(_IDX_CHUNK, rem)
        sub_sizes.append(s)
        rem -= s

    mesh = plsc.VectorSubcoreMesh(core_axis_name="c", subcore_axis_name="s")

    @functools.partial(
        pl.kernel,
        out_type=jax.ShapeDtypeStruct((batch * list_len,), jnp.float32),
        mesh=mesh,
        compiler_params=pltpu.CompilerParams(
            needs_layout_passes=False, use_tc_tiling_on_sc=False),
        scratch_types=[
            pltpu.VMEM((users_per_w, dim), jnp.float32),
            pltpu.VMEM((users_per_w * list_len,), jnp.int32),
            pltpu.VMEM((list_len, dim), jnp.float32),
            pltpu.VMEM((list_len, dim), jnp.float32),
            pltpu.VMEM((users_per_w * list_len,), jnp.float32),
            pltpu.SemaphoreType.DMA,
            pltpu.SemaphoreType.DMA,
            pltpu.SemaphoreType.DMA,
        ],
    )
    def kb(uemb_hbm, iid_hbm, itab_hbm, out_hbm,
           urows, iidx, rows0, rows1, scores, sem0, sem1, semu):
        rows = (rows0, rows1)
        sems = (sem0, sem1)
        wid = _worker_id()
        ubase = wid * users_per_w

        pltpu.sync_copy(
            uemb_hbm.at[pl.ds(ubase, users_per_w), :], urows)
        pltpu.sync_copy(
            iid_hbm.at[pl.ds(ubase * list_len, users_per_w * list_len)], iidx)

        def gather_descs(u, b):
            off = pl.multiple_of(u * list_len, 8)
            descs = []
            pos = 0
            for s in sub_sizes:
                descs.append(pltpu.make_async_copy(
                    itab_hbm.at[iidx.at[pl.ds(off + pos, s)]],
                    rows[b].at[pl.ds(pos, s)],
                    sems[b]))
                pos += s
            return descs

        def compute(u, rowsb):
            iota = lax.iota(jnp.int32, _LANES)
            u0 = urows[u, pl.ds(0, _LANES)]
            u1 = urows[u, pl.ds(_LANES, _LANES)]
            starts = _group_starts(list_len)
            ridxs = [iota + l0 for l0 in starts]
            accs = [jnp.zeros((_LANES,), jnp.float32) for _ in starts]
            for d in range(dim):
                colidx = (iota + d) & (dim - 1)
                lo = colidx & (_LANES - 1)
                urot = jnp.where(colidx < _LANES,
                                 _take16(u0, lo), _take16(u1, lo))
                for g in range(len(starts)):
                    v = plsc.load_gather(rowsb, [ridxs[g], colidx])
                    accs[g] = accs[g] + v * urot
            for g, l0 in enumerate(starts):
                scores[pl.ds(u * list_len + l0, _LANES)] = accs[g]

        for d in gather_descs(0, 0):
            d.start()

        def body(uu, carry):
            for b in range(2):
                u = uu * 2 + b
                for d in gather_descs(u, b):
                    d.wait()

                @pl.when(u + 1 < users_per_w)
                def _prefetch():
                    for d in gather_descs(u + 1, 1 - b):
                        d.start()

                compute(u, rows[b])
            return carry

        lax.fori_loop(0, users_per_w // 2, body, 0)
        pltpu.sync_copy(
            scores,
            out_hbm.at[pl.ds(ubase * list_len, users_per_w * list_len)])

    return kb


@functools.lru_cache(maxsize=None)
def _make_retile_kernel(batch, list_len):
    info = plsc.get_sparse_core_info()
    nw = info.num_cores * info.num_subcores
    users_per_w = batch // nw
    mesh = plsc.VectorSubcoreMesh(core_axis_name="c", subcore_axis_name="s")

    @functools.partial(
        pl.kernel,
        out_type=jax.ShapeDtypeStruct((list_len, batch), jnp.float32),
        mesh=mesh,
        compiler_params=pltpu.CompilerParams(needs_layout_passes=False),
        scratch_types=[
            pltpu.VMEM((users_per_w * list_len,), jnp.float32),
            pltpu.VMEM((list_len, users_per_w), jnp.float32),
            pltpu.SemaphoreType.DMA,
        ],
    )
    def kc(flat_hbm, out_hbm, sbuf, obuf, sem):
        wid = _worker_id()
        ubase = wid * users_per_w
        pltpu.async_copy(
            flat_hbm.at[pl.ds(ubase * list_len, users_per_w * list_len)],
            sbuf, sem).wait()

        def row(l, carry):
            for u0 in range(0, users_per_w, _LANES):
                idx = (lax.iota(jnp.int32, _LANES) + u0) * list_len + l
                v = plsc.load_gather(sbuf, [idx])
                obuf[l, pl.ds(u0, _LANES)] = v
            return carry

        lax.fori_loop(0, list_len, row, 0)
        pltpu.async_copy(
            obuf, out_hbm.at[:, pl.ds(ubase, users_per_w)], sem).wait()

    return kc


def kernel(user_id, item_ids, user_table, item_table):
    batch, list_len = item_ids.shape
    vocab, dim = item_table.shape
    ka = _make_format_kernel(batch, list_len, vocab, dim)
    it_flat, uemb_flat, ids_flat = ka(
        item_table.T, item_ids.T, user_id, user_table.T)
    kb = _make_score_kernel(batch, list_len, vocab, dim)
    scores_flat = kb(uemb_flat.reshape(batch, dim), ids_flat,
                     it_flat.reshape(vocab, dim))
    kc = _make_retile_kernel(batch, list_len)
    return kc(scores_flat).T


# speedup vs baseline: 1.1548x; 1.0064x over previous
import functools

import jax
import jax.numpy as jnp
from jax import lax
from jax.experimental import pallas as pl
from jax.experimental.pallas import tpu as pltpu
from jax.experimental.pallas import tpu_sc as plsc

_LANES = 16
_IDX_CHUNK = 128


def _worker_id():
    return lax.axis_index("s") * 2 + lax.axis_index("c")


def _take16(vec, idx):
    return lax.gather(
        vec, idx[:, None],
        dimension_numbers=lax.GatherDimensionNumbers(
            offset_dims=(), collapsed_slice_dims=(0,), start_index_map=(0,)),
        slice_sizes=(1,),
        mode=lax.GatherScatterMode.PROMISE_IN_BOUNDS)


def _group_starts(list_len):
    starts = list(range(0, list_len - _LANES + 1, _LANES))
    if starts[-1] + _LANES < list_len:
        starts.append(list_len - _LANES)
    return starts


@functools.lru_cache(maxsize=None)
def _make_format_kernel(batch, list_len, vocab, dim):
    info = plsc.get_sparse_core_info()
    nw = info.num_cores * info.num_subcores
    users_per_w = batch // nw
    n_tile_cols = vocab // _IDX_CHUNK
    vocab_tail = vocab - n_tile_cols * _IDX_CHUNK
    full_cols = n_tile_cols
    cb_w = _IDX_CHUNK * 2
    n_batch = full_cols // 2
    units = (n_batch // nw) & ~1
    rem_batches = n_batch - units * nw
    mesh = plsc.VectorSubcoreMesh(core_axis_name="c", subcore_axis_name="s")

    @functools.partial(
        pl.kernel,
        out_type=(
            jax.ShapeDtypeStruct((vocab * dim,), jnp.float32),
            jax.ShapeDtypeStruct((batch * dim,), jnp.float32),
            jax.ShapeDtypeStruct((batch * list_len,), jnp.int32),
        ),
        mesh=mesh,
        compiler_params=pltpu.CompilerParams(needs_layout_passes=False),
        scratch_types=[
            pltpu.VMEM((dim, cb_w), jnp.float32),
            pltpu.VMEM((dim, cb_w), jnp.float32),
            pltpu.VMEM((cb_w * dim,), jnp.float32),
            pltpu.VMEM((cb_w * dim,), jnp.float32),
            pltpu.VMEM((dim, _IDX_CHUNK), jnp.float32),
            pltpu.VMEM((dim, _IDX_CHUNK), jnp.float32),
            pltpu.VMEM((dim, vocab_tail), jnp.float32) if vocab_tail
            else pltpu.VMEM((dim, _IDX_CHUNK), jnp.float32),
            pltpu.VMEM((max(vocab_tail, 1) * dim,), jnp.float32),
            pltpu.VMEM((users_per_w * dim,), jnp.float32),
            pltpu.VMEM((list_len, _IDX_CHUNK), jnp.int32),
            pltpu.VMEM((users_per_w * list_len,), jnp.int32),
            pltpu.VMEM((users_per_w,), jnp.int32),
            pltpu.SemaphoreType.DMA,
            pltpu.SemaphoreType.DMA,
            pltpu.SemaphoreType.DMA,
            pltpu.SemaphoreType.DMA,
        ],
    )
    def ka(itab_t, ids_t, uid_hbm, utab_t,
           it_flat, uemb_flat, ids_flat,
           vb0, vb1, tb0, tb1, ub0, ub1, pb, ptb, ueb, ib, iob, uvm,
           si0, si1, so0, so1):
        vbs = (vb0, vb1)
        tbs = (tb0, tb1)
        ubs = (ub0, ub1)
        sis = (si0, si1)
        sos = (so0, so1)
        wid = _worker_id()
        rbase = wid * units * cb_w

        def in_desc(k, b):
            return pltpu.make_async_copy(
                itab_t.at[:, pl.ds(rbase + k * cb_w, cb_w)],
                vbs[b], sis[b])

        def out_desc(k, b):
            return pltpu.make_async_copy(
                tbs[b],
                it_flat.at[pl.ds((rbase + k * cb_w) * dim, cb_w * dim)],
                sos[b])

        iota = lax.iota(jnp.int32, _LANES)
        perms = [((iota + s) & (_LANES - 1)) for s in range(_LANES)]

        def transpose_block(src, dst, n_cols):
            def cols16(i, carry):
                colidx = iota + i * _LANES
                base = colidx * dim
                pairs = []
                for d0 in range(0, dim, _LANES):
                    for s in range(_LANES):
                        rowidx = perms[s] + d0
                        v = plsc.load_gather(src, [rowidx, colidx])
                        pairs.append((base + rowidx, v))
                for idx, v in pairs:
                    plsc.store_scatter(dst, [idx], v)
                return carry
            lax.fori_loop(0, n_cols // _LANES, cols16, 0)

        in_desc(0, 0).start()

        def col_body(k2, carry):
            for b in range(2):
                k = k2 * 2 + b
                in_desc(k, b).wait()

                @pl.when(k + 1 < units)
                def _pref():
                    in_desc(k + 1, 1 - b).start()

                @pl.when(k >= 2)
                def _drain():
                    out_desc(k - 2, b).wait()

                transpose_block(vbs[b], tbs[b], cb_w)
                out_desc(k, b).start()
            return carry

        lax.fori_loop(0, units // 2, col_body, 0)
        out_desc(units - 2, 0).wait()
        out_desc(units - 1, 1).wait()

        @pl.when(wid == nw - 1)
        def _tail():
            r0 = nw * units * cb_w
            for j in range(rem_batches):
                pltpu.async_copy(
                    itab_t.at[:, pl.ds(r0 + j * cb_w, cb_w)], vb0, si0).wait()
                transpose_block(vb0, tb0, cb_w)
                pltpu.async_copy(
                    tb0, it_flat.at[pl.ds((r0 + j * cb_w) * dim, cb_w * dim)],
                    so0).wait()
            if vocab_tail:
                pltpu.async_copy(
                    itab_t.at[:, pl.ds(n_tile_cols * _IDX_CHUNK, vocab_tail)],
                    pb, si0).wait()
                transpose_block(pb, ptb, vocab_tail)
                pltpu.async_copy(
                    ptb,
                    it_flat.at[pl.ds(n_tile_cols * _IDX_CHUNK * dim,
                                     vocab_tail * dim)],
                    so0).wait()

        ubase = wid * users_per_w
        pltpu.sync_copy(uid_hbm.at[pl.ds(ubase, users_per_w)], uvm)

        def ufetch_desc(uid_scalar, b):
            col = pl.multiple_of((uid_scalar // _IDX_CHUNK) * _IDX_CHUNK,
                                 _IDX_CHUNK)
            return pltpu.make_async_copy(
                utab_t.at[:, pl.ds(col, _IDX_CHUNK)], ubs[b], sis[b])

        def user_body(u2, carry):
            uvec = uvm[pl.ds(u2 * _LANES, _LANES)]
            uids = [uvec[j] for j in range(_LANES)]
            ufetch_desc(uids[0], 0).start()
            for j in range(_LANES):
                b = j % 2
                ufetch_desc(uids[j], b).wait()
                if j + 1 < _LANES:
                    ufetch_desc(uids[j + 1], 1 - b).start()
                lane = uids[j] % _IDX_CHUNK
                u = u2 * _LANES + j
                for d0 in range(0, dim, _LANES):
                    ridx = lax.iota(jnp.int32, _LANES) + d0
                    cidx = jnp.full((_LANES,), lane, jnp.int32)
                    v = plsc.load_gather(ubs[b], [ridx, cidx])
                    ueb[pl.ds(u * dim + d0, _LANES)] = v
            return carry

        lax.fori_loop(0, users_per_w // _LANES, user_body, 0)
        pltpu.sync_copy(ueb, uemb_flat.at[pl.ds(ubase * dim,
                                                users_per_w * dim)])

        pltpu.sync_copy(ids_t.at[:, pl.ds(ubase, users_per_w)], ib)

        def ids_body(u, carry):
            for l0 in _group_starts(list_len):
                ridx = lax.iota(jnp.int32, _LANES) + l0
                cidx = jnp.full((_LANES,), u, jnp.int32)
                v = plsc.load_gather(ib, [ridx, cidx])
                iob[pl.ds(u * list_len + l0, _LANES)] = v
            return carry

        lax.fori_loop(0, users_per_w, ids_body, 0)
        pltpu.sync_copy(
            iob, ids_flat.at[pl.ds(ubase * list_len, users_per_w * list_len)])

    return ka


@functools.lru_cache(maxsize=None)
def _make_score_kernel(batch, list_len, vocab, dim):
    info = plsc.get_sparse_core_info()
    nw = info.num_cores * info.num_subcores
    users_per_w = batch // nw
    sub_sizes = [list_len]

    mesh = plsc.VectorSubcoreMesh(core_axis_name="c", subcore_axis_name="s")

    @functools.partial(
        pl.kernel,
        out_type=jax.ShapeDtypeStruct((batch * list_len,), jnp.float32),
        mesh=mesh,
        compiler_params=pltpu.CompilerParams(
            needs_layout_passes=False, use_tc_tiling_on_sc=False),
        scratch_types=[
            pltpu.VMEM((users_per_w, dim), jnp.float32),
            pltpu.VMEM((users_per_w * list_len,), jnp.int32),
            pltpu.VMEM((list_len, dim), jnp.float32),
            pltpu.VMEM((list_len, dim), jnp.float32),
            pltpu.VMEM((users_per_w * list_len,), jnp.float32),
            pltpu.SemaphoreType.DMA,
            pltpu.SemaphoreType.DMA,
            pltpu.SemaphoreType.DMA,
        ],
    )
    def kb(uemb_hbm, iid_hbm, itab_hbm, out_hbm,
           urows, iidx, rows0, rows1, scores, sem0, sem1, semu):
        rows = (rows0, rows1)
        sems = (sem0, sem1)
        wid = _worker_id()
        ubase = wid * users_per_w

        pltpu.sync_copy(
            uemb_hbm.at[pl.ds(ubase, users_per_w), :], urows)
        pltpu.sync_copy(
            iid_hbm.at[pl.ds(ubase * list_len, users_per_w * list_len)], iidx)

        def gather_descs(u, b):
            off = pl.multiple_of(u * list_len, 8)
            descs = []
            pos = 0
            for s in sub_sizes:
                descs.append(pltpu.make_async_copy(
                    itab_hbm.at[iidx.at[pl.ds(off + pos, s)]],
                    rows[b].at[pl.ds(pos, s)],
                    sems[b]))
                pos += s
            return descs

        def compute(u, rowsb):
            iota = lax.iota(jnp.int32, _LANES)
            u0 = urows[u, pl.ds(0, _LANES)]
            u1 = urows[u, pl.ds(_LANES, _LANES)]
            starts = _group_starts(list_len)
            ridxs = [iota + l0 for l0 in starts]
            accs = [jnp.zeros((_LANES,), jnp.float32) for _ in starts]
            for d in range(dim):
                colidx = (iota + d) & (dim - 1)
                lo = colidx & (_LANES - 1)
                urot = jnp.where(colidx < _LANES,
                                 _take16(u0, lo), _take16(u1, lo))
                for g in range(len(starts)):
                    v = plsc.load_gather(rowsb, [ridxs[g], colidx])
                    accs[g] = accs[g] + v * urot
            for g, l0 in enumerate(starts):
                scores[pl.ds(u * list_len + l0, _LANES)] = accs[g]

        for d in gather_descs(0, 0):
            d.start()

        def body(uu, carry):
            for b in range(2):
                u = uu * 2 + b
                for d in gather_descs(u, b):
                    d.wait()

                @pl.when(u + 1 < users_per_w)
                def _prefetch():
                    for d in gather_descs(u + 1, 1 - b):
                        d.start()

                compute(u, rows[b])
            return carry

        lax.fori_loop(0, users_per_w // 2, body, 0)
        pltpu.sync_copy(
            scores,
            out_hbm.at[pl.ds(ubase * list_len, users_per_w * list_len)])

    return kb


@functools.lru_cache(maxsize=None)
def _make_retile_kernel(batch, list_len):
    info = plsc.get_sparse_core_info()
    nw = info.num_cores * info.num_subcores
    users_per_w = batch // nw
    mesh = plsc.VectorSubcoreMesh(core_axis_name="c", subcore_axis_name="s")

    @functools.partial(
        pl.kernel,
        out_type=jax.ShapeDtypeStruct((list_len, batch), jnp.float32),
        mesh=mesh,
        compiler_params=pltpu.CompilerParams(needs_layout_passes=False),
        scratch_types=[
            pltpu.VMEM((users_per_w * list_len,), jnp.float32),
            pltpu.VMEM((list_len, users_per_w), jnp.float32),
            pltpu.SemaphoreType.DMA,
        ],
    )
    def kc(flat_hbm, out_hbm, sbuf, obuf, sem):
        wid = _worker_id()
        ubase = wid * users_per_w
        pltpu.async_copy(
            flat_hbm.at[pl.ds(ubase * list_len, users_per_w * list_len)],
            sbuf, sem).wait()

        def row(l, carry):
            for u0 in range(0, users_per_w, _LANES):
                idx = (lax.iota(jnp.int32, _LANES) + u0) * list_len + l
                v = plsc.load_gather(sbuf, [idx])
                obuf[l, pl.ds(u0, _LANES)] = v
            return carry

        lax.fori_loop(0, list_len, row, 0)
        pltpu.async_copy(
            obuf, out_hbm.at[:, pl.ds(ubase, users_per_w)], sem).wait()

    return kc


def kernel(user_id, item_ids, user_table, item_table):
    batch, list_len = item_ids.shape
    vocab, dim = item_table.shape
    ka = _make_format_kernel(batch, list_len, vocab, dim)
    it_flat, uemb_flat, ids_flat = ka(
        item_table.T, item_ids.T, user_id, user_table.T)
    kb = _make_score_kernel(batch, list_len, vocab, dim)
    scores_flat = kb(uemb_flat.reshape(batch, dim), ids_flat,
                     it_flat.reshape(vocab, dim))
    kc = _make_retile_kernel(batch, list_len)
    return kc(scores_flat).T
